# baseline (speedup 1.0000x reference)
import functools

import jax
import jax.numpy as jnp
from jax import lax
from jax.experimental import pallas as pl
from jax.experimental.pallas import tpu as pltpu
from jax.experimental.pallas import tpu_sc as plsc

PAGE = 128
NRING = 6


def kernel(pos_ids, k_val, v_val, batch_idx, k_cache, v_cache, page_table):
    B, H, S, D = k_val.shape
    T = k_cache.shape[0]
    NP = T // PAGE
    LP = S // PAGE
    NWT = B * LP
    NPT = NP - NWT

    info = plsc.get_sparse_core_info()
    NC, NS, L = info.num_cores, info.num_subcores, info.num_lanes
    NWK = NC * NS
    WPW = NWT // NWK
    PPW = NPT // NWK
    RPP = PAGE * H

    lp0 = pos_ids.astype(jnp.int32)[0, ::PAGE] >> 7
    dp = page_table[batch_idx.astype(jnp.int32)[:, None], lp0[None, :]]
    dp_flat = dp.reshape(-1)
    mark = jnp.zeros((NP,), jnp.int32).at[dp_flat].set(1)
    unt = jnp.argsort(mark, stable=True)[:NPT].astype(jnp.int32)

    ar = jnp.arange(RPP, dtype=jnp.int32)
    j_tok, h_head = ar % PAGE, ar // PAGE
    wt = jnp.arange(NWT, dtype=jnp.int32)
    bsrc, slot = wt // LP, wt % LP
    wr_g = (bsrc * (H * S) + slot * PAGE)[:, None] + (h_head * S + j_tok)[None, :]
    wr_o = (dp_flat * RPP)[:, None] + (j_tok * H + h_head)[None, :]
    pa = (unt * RPP)[:, None] + ar[None, :]
    wr_g = wr_g.reshape(NWK, WPW * 8, PAGE)
    wr_o = wr_o.reshape(NWK, WPW * 8, PAGE)
    pa = pa.reshape(NWK, PPW * 8, PAGE)

    kvr = k_val.reshape(B * H * S, D)
    vvr = v_val.reshape(B * H * S, D)
    kcr = k_cache.reshape(T * H, D)
    vcr = v_cache.reshape(T * H, D)

    mesh = plsc.VectorSubcoreMesh(core_axis_name="c", subcore_axis_name="s")

    @functools.partial(
        pl.kernel, mesh=mesh,
        out_type=[jax.ShapeDtypeStruct((T * H, D), k_cache.dtype),
                  jax.ShapeDtypeStruct((T * H, D), v_cache.dtype)],
        scratch_types=[
            pltpu.VMEM((WPW * 8, PAGE), jnp.int32),
            pltpu.VMEM((WPW * 8, PAGE), jnp.int32),
            pltpu.VMEM((PPW * 8, PAGE), jnp.int32),
            pltpu.VMEM((NRING * PAGE, D), jnp.float32),
            [pltpu.SemaphoreType.DMA] * NRING,
            [pltpu.SemaphoreType.DMA] * NRING,
        ],
    )
    def sc_fill(wrg_hbm, wro_hbm, pa_hbm, kv_hbm, vv_hbm, kc_hbm, vc_hbm,
                ko_hbm, vo_hbm, gix, oix, pix, buf, gsems, ssems):
        wid = lax.axis_index("s") * NC + lax.axis_index("c")
        slots = [buf.at[pl.ds(q * PAGE, PAGE)] for q in range(NRING)]
        pltpu.sync_copy(wrg_hbm.at[wid], gix)
        pltpu.sync_copy(wro_hbm.at[wid], oix)
        pltpu.sync_copy(pa_hbm.at[wid], pix)

        chunks = []
        for m in range(WPW * 8):
            chunks.append((kv_hbm, ko_hbm, gix.at[m], oix.at[m]))
        for m in range(PPW * 8):
            chunks.append((kc_hbm, ko_hbm, pix.at[m], pix.at[m]))
        for m in range(WPW * 8):
            chunks.append((vv_hbm, vo_hbm, gix.at[m], oix.at[m]))
        for m in range(PPW * 8):
            chunks.append((vc_hbm, vo_hbm, pix.at[m], pix.at[m]))

        n = len(chunks)
        pend_g = [None] * NRING
        pend_s = [None] * NRING
        for m in range(n):
            q = m % NRING
            if pend_s[q] is not None:
                pend_s[q].wait()
            src, _, gr, _ = chunks[m]
            pend_g[q] = pltpu.async_copy(src.at[gr], slots[q], gsems[q])
            if m >= 1:
                qp = (m - 1) % NRING
                _, out, _, orow = chunks[m - 1]
                pend_g[qp].wait()
                pend_s[qp] = pltpu.async_copy(slots[qp], out.at[orow],
                                              ssems[qp])
        qp = (n - 1) % NRING
        _, out, _, orow = chunks[n - 1]
        pend_g[qp].wait()
        pend_s[qp] = pltpu.async_copy(slots[qp], out.at[orow], ssems[qp])
        for q in range(NRING):
            if pend_s[q] is not None:
                pend_s[q].wait()

    ko, vo = sc_fill(wr_g, wr_o, pa, kvr, vvr, kcr, vcr)
    return ko.reshape(T, H, D), vo.reshape(T, H, D)


# speedup vs baseline: 1.0060x; 1.0060x over previous
import functools

import jax
import jax.numpy as jnp
from jax import lax
from jax.experimental import pallas as pl
from jax.experimental.pallas import tpu as pltpu
from jax.experimental.pallas import tpu_sc as plsc

PAGE = 128
NRING = 6


def kernel(pos_ids, k_val, v_val, batch_idx, k_cache, v_cache, page_table):
    B, H, S, D = k_val.shape
    T = k_cache.shape[0]
    NP = T // PAGE
    LP = S // PAGE
    NWT = B * LP
    NPT = NP - NWT

    info = plsc.get_sparse_core_info()
    NC, NS, L = info.num_cores, info.num_subcores, info.num_lanes
    NWK = NC * NS
    WPW = NWT // NWK
    PPW = NPT // NWK
    RPP = PAGE * H

    lp0 = pos_ids.astype(jnp.int32)[0, ::PAGE] >> 7
    dp = page_table[batch_idx.astype(jnp.int32)[:, None], lp0[None, :]]
    dp_flat = dp.reshape(-1)
    mark = jnp.zeros((NP,), jnp.int32).at[dp_flat].set(1)
    unt = jnp.argsort(mark, stable=True)[:NPT].astype(jnp.int32)

    ar = jnp.arange(RPP, dtype=jnp.int32)
    j_tok, h_head = ar // H, ar % H
    wt = jnp.arange(NWT, dtype=jnp.int32)
    bsrc, slot = wt // LP, wt % LP
    wr_g = (bsrc * (H * S) + slot * PAGE)[:, None] + (h_head * S + j_tok)[None, :]
    wr_o = (dp_flat * RPP)[:, None] + ar[None, :]
    pa = (unt * RPP)[:, None] + ar[None, :]
    wr_g = wr_g.reshape(NWK, WPW * 8, PAGE)
    wr_o = wr_o.reshape(NWK, WPW * 8, PAGE)
    pa = pa.reshape(NWK, PPW * 8, PAGE)

    kvr = k_val.reshape(B * H * S, D)
    vvr = v_val.reshape(B * H * S, D)
    kcr = k_cache.reshape(T * H, D)
    vcr = v_cache.reshape(T * H, D)

    mesh = plsc.VectorSubcoreMesh(core_axis_name="c", subcore_axis_name="s")

    @functools.partial(
        pl.kernel, mesh=mesh,
        out_type=[jax.ShapeDtypeStruct((T * H, D), k_cache.dtype),
                  jax.ShapeDtypeStruct((T * H, D), v_cache.dtype)],
        scratch_types=[
            pltpu.VMEM((WPW * 8, PAGE), jnp.int32),
            pltpu.VMEM((WPW * 8, PAGE), jnp.int32),
            pltpu.VMEM((PPW * 8, PAGE), jnp.int32),
            pltpu.VMEM((NRING * PAGE, D), jnp.float32),
            [pltpu.SemaphoreType.DMA] * NRING,
            [pltpu.SemaphoreType.DMA] * NRING,
        ],
    )
    def sc_fill(wrg_hbm, wro_hbm, pa_hbm, kv_hbm, vv_hbm, kc_hbm, vc_hbm,
                ko_hbm, vo_hbm, gix, oix, pix, buf, gsems, ssems):
        wid = lax.axis_index("s") * NC + lax.axis_index("c")
        slots = [buf.at[pl.ds(q * PAGE, PAGE)] for q in range(NRING)]
        pltpu.sync_copy(wrg_hbm.at[wid], gix)
        pltpu.sync_copy(wro_hbm.at[wid], oix)
        pltpu.sync_copy(pa_hbm.at[wid], pix)

        chunks = []
        for m in range(WPW * 8):
            chunks.append((kv_hbm, ko_hbm, gix.at[m], oix.at[m]))
        for m in range(PPW * 8):
            chunks.append((kc_hbm, ko_hbm, pix.at[m], pix.at[m]))
        for m in range(WPW * 8):
            chunks.append((vv_hbm, vo_hbm, gix.at[m], oix.at[m]))
        for m in range(PPW * 8):
            chunks.append((vc_hbm, vo_hbm, pix.at[m], pix.at[m]))

        n = len(chunks)
        pend_g = [None] * NRING
        pend_s = [None] * NRING
        for m in range(n):
            q = m % NRING
            if pend_s[q] is not None:
                pend_s[q].wait()
            src, _, gr, _ = chunks[m]
            pend_g[q] = pltpu.async_copy(src.at[gr], slots[q], gsems[q])
            if m >= 1:
                qp = (m - 1) % NRING
                _, out, _, orow = chunks[m - 1]
                pend_g[qp].wait()
                pend_s[qp] = pltpu.async_copy(slots[qp], out.at[orow],
                                              ssems[qp])
        qp = (n - 1) % NRING
        _, out, _, orow = chunks[n - 1]
        pend_g[qp].wait()
        pend_s[qp] = pltpu.async_copy(slots[qp], out.at[orow], ssems[qp])
        for q in range(NRING):
            if pend_s[q] is not None:
                pend_s[q].wait()

    ko, vo = sc_fill(wr_g, wr_o, pa, kvr, vvr, kcr, vcr)
    return ko.reshape(T, H, D), vo.reshape(T, H, D)
